# baseline (speedup 1.0000x reference)
import functools

import jax
import jax.numpy as jnp
from jax import lax
from jax.experimental import pallas as pl
from jax.experimental.pallas import tpu as pltpu
from jax.experimental.pallas import tpu_sc as plsc

B = 16384
DIM = 32
NBUCKETS = 1000
OUT_ROWS = 2 * DIM + 1

NC = 2
NS = 16
L = 16
NW = NC * NS
BPW = B // NW
NVEC = BPW // L
BKT_PAD = 1024
CONST_PAD = 128
TS_PAD = 1008


def _sc_body(uid_hbm, ts_hbm, utabT_hbm, ttabT_hbm, bkt_hbm, consts_hbm,
             out_hbm, uid_v, ts_v, idx_v, bkt_v, consts_v, ur_v, tr_v,
             nrm_v, sem_u, sem_t):
    wid = lax.axis_index("s") * NC + lax.axis_index("c")
    base = wid * BPW

    pltpu.sync_copy(uid_hbm.at[pl.ds(base, BPW)], uid_v)
    user_copies = [
        pltpu.async_copy(utabT_hbm.at[d].at[uid_v], ur_v.at[d], sem_u)
        for d in range(DIM)
    ]

    pltpu.sync_copy(ts_hbm.at[pl.ds(base, BPW)], ts_v)
    pltpu.sync_copy(bkt_hbm, bkt_v)
    pltpu.sync_copy(consts_hbm, consts_v)

    inv_step = consts_v[pl.ds(0, L)]
    mean = consts_v[pl.ds(L, L)]
    denom = consts_v[pl.ds(2 * L, L)]

    for i in range(NVEC):
        t = ts_v[pl.ds(i * L, L)]
        g = jnp.clip((t * inv_step).astype(jnp.int32), 0, NBUCKETS - 2)
        blo = plsc.load_gather(bkt_v, [g])
        bhi = plsc.load_gather(bkt_v, [g + 1])
        idx = jnp.where(t < blo, g, jnp.where(t >= bhi, g + 2, g + 1))
        idx_v[pl.ds(i * L, L)] = idx
        nrm_v[pl.ds(i * L, L)] = (t - mean) / denom

    ts_copies = [
        pltpu.async_copy(ttabT_hbm.at[d].at[idx_v], tr_v.at[d], sem_t)
        for d in range(DIM)
    ]

    pltpu.sync_copy(nrm_v, out_hbm.at[2 * DIM, pl.ds(base, BPW)])
    for c in user_copies:
        c.wait()
    pltpu.sync_copy(ur_v, out_hbm.at[pl.ds(0, DIM), pl.ds(base, BPW)])
    for c in ts_copies:
        c.wait()
    pltpu.sync_copy(tr_v, out_hbm.at[pl.ds(DIM, DIM), pl.ds(base, BPW)])


@jax.jit
def _run(user_id, timestamp, utabT, ttabT, buckets_pad, consts):
    mesh = plsc.VectorSubcoreMesh(core_axis_name="c", subcore_axis_name="s")
    f = functools.partial(
        pl.kernel,
        mesh=mesh,
        compiler_params=pltpu.CompilerParams(
            needs_layout_passes=False, use_tc_tiling_on_sc=False),
        out_type=jax.ShapeDtypeStruct((OUT_ROWS, B), jnp.float32),
        scratch_types=[
            pltpu.VMEM((BPW,), jnp.int32),
            pltpu.VMEM((BPW,), jnp.float32),
            pltpu.VMEM((BPW,), jnp.int32),
            pltpu.VMEM((BKT_PAD,), jnp.float32),
            pltpu.VMEM((CONST_PAD,), jnp.float32),
            pltpu.VMEM((DIM, BPW), jnp.float32),
            pltpu.VMEM((DIM, BPW), jnp.float32),
            pltpu.VMEM((BPW,), jnp.float32),
            pltpu.SemaphoreType.DMA,
            pltpu.SemaphoreType.DMA,
        ],
    )(_sc_body)
    return f(user_id, timestamp, utabT, ttabT, buckets_pad, consts)


def kernel(user_id, timestamp, user_table, ts_table, buckets, norm_mean,
           norm_var):
    n = buckets.shape[0]
    inv_step = (n - 1.0) / (buckets[-1] - buckets[0])
    denom = jnp.sqrt(norm_var + 1e-6)
    consts = jnp.concatenate([
        jnp.full((L,), inv_step, jnp.float32),
        jnp.full((L,), norm_mean, jnp.float32),
        jnp.full((L,), denom, jnp.float32),
        jnp.zeros((CONST_PAD - 3 * L,), jnp.float32),
    ])
    buckets_pad = jnp.concatenate(
        [buckets, jnp.full((BKT_PAD - n,), jnp.inf, jnp.float32)])
    utabT = user_table.T
    ttabT = jnp.pad(ts_table.T, ((0, 0), (0, TS_PAD - ts_table.shape[0])))
    outT = _run(user_id, timestamp, utabT, ttabT, buckets_pad, consts)
    return outT.T


# speedup vs baseline: 4.9027x; 4.9027x over previous
import functools

import jax
import jax.numpy as jnp
from jax import lax
from jax.experimental import pallas as pl
from jax.experimental.pallas import tpu as pltpu
from jax.experimental.pallas import tpu_sc as plsc

B = 16384
DIM = 32
NBUCKETS = 1000
OUT_COLS = 2 * DIM + 1

NC = 2
NS = 16
L = 16
NW = NC * NS
BPW = B // NW
NVEC = BPW // L
BKT_PAD = 1024
CONST_PAD = 128


def _sc_body(uid_hbm, ts_hbm, utab_hbm, ttab_hbm, bkt_hbm, consts_hbm,
             out_hbm, uid_v, ts_v, idx_v, bkt_v, consts_v, urows_v, trows_v,
             nrm_v, sem_u, sem_t):
    wid = lax.axis_index("s") * NC + lax.axis_index("c")
    base = wid * BPW

    pltpu.sync_copy(uid_hbm.at[pl.ds(base, BPW)], uid_v)
    user_gather = pltpu.async_copy(utab_hbm.at[uid_v], urows_v, sem_u)

    pltpu.sync_copy(ts_hbm.at[pl.ds(base, BPW)], ts_v)
    pltpu.sync_copy(bkt_hbm, bkt_v)
    pltpu.sync_copy(consts_hbm, consts_v)

    inv_step = consts_v[pl.ds(0, L)]
    mean = consts_v[pl.ds(L, L)]
    denom = consts_v[pl.ds(2 * L, L)]
    lanes = lax.iota(jnp.int32, L)
    zeros = jnp.zeros((L,), jnp.int32)

    for i in range(NVEC):
        t = ts_v[pl.ds(i * L, L)]
        g = jnp.clip((t * inv_step).astype(jnp.int32), 0, NBUCKETS - 2)
        blo = plsc.load_gather(bkt_v, [g])
        bhi = plsc.load_gather(bkt_v, [g + 1])
        idx = jnp.where(t < blo, g, jnp.where(t >= bhi, g + 2, g + 1))
        idx_v[pl.ds(i * L, L)] = idx
        plsc.store_scatter(nrm_v, [i * L + lanes, zeros], (t - mean) / denom)

    ts_gather = pltpu.async_copy(ttab_hbm.at[idx_v], trows_v, sem_t)

    user_gather.wait()
    pltpu.sync_copy(urows_v, out_hbm.at[pl.ds(base, BPW), pl.ds(0, DIM)])
    ts_gather.wait()
    pltpu.sync_copy(trows_v, out_hbm.at[pl.ds(base, BPW), pl.ds(DIM, DIM)])
    pltpu.sync_copy(nrm_v, out_hbm.at[pl.ds(base, BPW), pl.ds(2 * DIM, 1)])


@jax.jit
def _run(user_id, timestamp, user_table, ts_table, buckets_pad, consts):
    mesh = plsc.VectorSubcoreMesh(core_axis_name="c", subcore_axis_name="s")
    f = functools.partial(
        pl.kernel,
        mesh=mesh,
        compiler_params=pltpu.CompilerParams(
            needs_layout_passes=False, use_tc_tiling_on_sc=False),
        out_type=jax.ShapeDtypeStruct((B, OUT_COLS), jnp.float32),
        scratch_types=[
            pltpu.VMEM((BPW,), jnp.int32),
            pltpu.VMEM((BPW,), jnp.float32),
            pltpu.VMEM((BPW,), jnp.int32),
            pltpu.VMEM((BKT_PAD,), jnp.float32),
            pltpu.VMEM((CONST_PAD,), jnp.float32),
            pltpu.VMEM((BPW, DIM), jnp.float32),
            pltpu.VMEM((BPW, DIM), jnp.float32),
            pltpu.VMEM((BPW, 1), jnp.float32),
            pltpu.SemaphoreType.DMA,
            pltpu.SemaphoreType.DMA,
        ],
    )(_sc_body)
    return f(user_id, timestamp, user_table, ts_table, buckets_pad, consts)


def kernel(user_id, timestamp, user_table, ts_table, buckets, norm_mean,
           norm_var):
    n = buckets.shape[0]
    inv_step = (n - 1.0) / (buckets[-1] - buckets[0])
    denom = jnp.sqrt(norm_var + 1e-6)
    consts = jnp.concatenate([
        jnp.full((L,), inv_step, jnp.float32),
        jnp.full((L,), norm_mean, jnp.float32),
        jnp.full((L,), denom, jnp.float32),
        jnp.zeros((CONST_PAD - 3 * L,), jnp.float32),
    ])
    buckets_pad = jnp.concatenate(
        [buckets, jnp.full((BKT_PAD - n,), jnp.inf, jnp.float32)])
    return _run(user_id, timestamp, user_table, ts_table, buckets_pad,
                consts)


# speedup vs baseline: 9.8296x; 2.0049x over previous
import functools

import jax
import jax.numpy as jnp
from jax import lax
from jax.experimental import pallas as pl
from jax.experimental.pallas import tpu as pltpu
from jax.experimental.pallas import tpu_sc as plsc

B = 16384
VOCAB = 1000000
DIM = 32
NBUCKETS = 1000
OUT_COLS = 2 * DIM + 1
OUT_PAD = 128

NC = 2
NS = 16
L = 16
NW = NC * NS
BPW = B // NW
NVEC = BPW // L
BKT_PAD = 1024
CONST_PAD = 128
K = 4

NTILE = VOCAB // 128
TAIL_BASE = NTILE * 128
TAIL_N = VOCAB - TAIL_BASE
TS_ROWS_PAD = 1024


def _sc_body(uid_hbm, ts_hbm, tabT_hbm, tail_hbm, tts_hbm, bkt_hbm,
             consts_hbm, out_hbm, uid_v, ts_v, tidx_v, bkt_v, consts_v,
             tail_v, tts_v, blk0, blk1, blk2, blk3, stage_v, sem):
    wid = lax.axis_index("s") * NC + lax.axis_index("c")
    base = wid * BPW
    blks = [blk0, blk1, blk2, blk3]
    lanes = lax.iota(jnp.int32, L)

    pltpu.sync_copy(uid_hbm.at[pl.ds(base, BPW)], uid_v)
    pltpu.sync_copy(ts_hbm.at[pl.ds(base, BPW)], ts_v)
    pltpu.sync_copy(bkt_hbm, bkt_v)
    pltpu.sync_copy(consts_hbm, consts_v)
    pltpu.sync_copy(tail_hbm, tail_v)
    pltpu.sync_copy(tts_hbm, tts_v)

    inv_step = consts_v[pl.ds(0, L)]
    mean = consts_v[pl.ds(L, L)]
    denom = consts_v[pl.ds(2 * L, L)]

    for i in range(NVEC):
        t = ts_v[pl.ds(i * L, L)]
        g = jnp.clip((t * inv_step).astype(jnp.int32), 0, NBUCKETS - 2)
        blo = plsc.load_gather(bkt_v, [g])
        bhi = plsc.load_gather(bkt_v, [g + 1])
        idx = jnp.where(t < blo, g, jnp.where(t >= bhi, g + 2, g + 1))
        tidx_v[pl.ds(i * L, L)] = idx
        plsc.store_scatter(stage_v, [i * L + lanes,
                                     jnp.full((L,), 2 * DIM, jnp.int32)],
                           (t - mean) / denom)

    for i in range(NVEC):
        tidx = tidx_v[pl.ds(i * L, L)]
        rows = i * L + lanes
        for d in range(DIM):
            vals = plsc.load_gather(tts_v, [tidx * DIM + d])
            plsc.store_scatter(stage_v, [rows,
                                         jnp.full((L,), DIM + d, jnp.int32)],
                               vals)

    def uscalar(j):
        voff = pl.multiple_of((j >> 4) * L, L)
        vec = uid_v[pl.ds(voff, L)]
        return jnp.sum(jnp.where(lanes == (j & (L - 1)), vec, 0))

    def group(gi, carry):
        j0 = gi * K
        for k in range(K):
            u = uscalar(j0 + k)
            utile = jnp.minimum(u >> 7, NTILE - 1)
            off = pl.multiple_of(utile * 128, 128)
            pltpu.async_copy(tabT_hbm.at[:, pl.ds(off, 128)], blks[k], sem)
        for k in range(K):
            pltpu.make_async_copy(tabT_hbm.at[:, pl.ds(0, 128)],
                                  blks[k], sem).wait()
        for k in range(K):
            j = j0 + k
            u = uscalar(j)
            ucol = jnp.full((L,), u & 127, jnp.int32)
            istail = u >= TAIL_BASE
            uloc = jnp.clip(u - TAIL_BASE, 0, TAIL_N - 1)
            for c in range(DIM // L):
                gn = plsc.load_gather(blks[k], [c * L + lanes, ucol])
                toff = pl.multiple_of(uloc * DIM + c * L, L)
                gt = tail_v[pl.ds(toff, L)]
                stage_v[j, pl.ds(c * L, L)] = jnp.where(istail, gt, gn)
        return carry

    lax.fori_loop(0, BPW // K, group, 0)

    pltpu.sync_copy(stage_v, out_hbm.at[pl.ds(base, BPW)])


@jax.jit
def _run(user_id, timestamp, tabT, tail, tts, buckets_pad, consts):
    mesh = plsc.VectorSubcoreMesh(core_axis_name="c", subcore_axis_name="s")
    f = functools.partial(
        pl.kernel,
        mesh=mesh,
        compiler_params=pltpu.CompilerParams(needs_layout_passes=False),
        out_type=jax.ShapeDtypeStruct((B, OUT_PAD), jnp.float32),
        scratch_types=[
            pltpu.VMEM((BPW,), jnp.int32),
            pltpu.VMEM((BPW,), jnp.float32),
            pltpu.VMEM((BPW,), jnp.int32),
            pltpu.VMEM((BKT_PAD,), jnp.float32),
            pltpu.VMEM((CONST_PAD,), jnp.float32),
            pltpu.VMEM((TAIL_N * DIM,), jnp.float32),
            pltpu.VMEM((TS_ROWS_PAD * DIM,), jnp.float32),
            pltpu.VMEM((DIM, 128), jnp.float32),
            pltpu.VMEM((DIM, 128), jnp.float32),
            pltpu.VMEM((DIM, 128), jnp.float32),
            pltpu.VMEM((DIM, 128), jnp.float32),
            pltpu.VMEM((BPW, OUT_PAD), jnp.float32),
            pltpu.SemaphoreType.DMA,
        ],
    )(_sc_body)
    return f(user_id, timestamp, tabT, tail, tts, buckets_pad, consts)


def kernel(user_id, timestamp, user_table, ts_table, buckets, norm_mean,
           norm_var):
    n = buckets.shape[0]
    inv_step = (n - 1.0) / (buckets[-1] - buckets[0])
    denom = jnp.sqrt(norm_var + 1e-6)
    consts = jnp.concatenate([
        jnp.full((L,), inv_step, jnp.float32),
        jnp.full((L,), norm_mean, jnp.float32),
        jnp.full((L,), denom, jnp.float32),
        jnp.zeros((CONST_PAD - 3 * L,), jnp.float32),
    ])
    buckets_pad = jnp.concatenate(
        [buckets, jnp.full((BKT_PAD - n,), jnp.inf, jnp.float32)])
    tabT = user_table.T
    tail = user_table[TAIL_BASE:].reshape(-1)
    tts = jnp.pad(
        ts_table, ((0, TS_ROWS_PAD - ts_table.shape[0]), (0, 0))).reshape(-1)
    outp = _run(user_id, timestamp, tabT, tail, tts, buckets_pad, consts)
    return outp[:, :OUT_COLS]


# speedup vs baseline: 14.8166x; 1.5073x over previous
import functools

import jax
import jax.numpy as jnp
from jax import lax
from jax.experimental import pallas as pl
from jax.experimental.pallas import tpu as pltpu
from jax.experimental.pallas import tpu_sc as plsc

B = 16384
VOCAB = 1000000
DIM = 32
NBUCKETS = 1000
OUT_COLS = 2 * DIM + 1
OUT_PAD = 128

NC = 2
NS = 16
L = 16
NW = NC * NS
BPW = B // NW
HALF = BPW // 2
HVEC = HALF // L
BKT_PAD = 1024
CONST_PAD = 128
K = 4
NGRP = HALF // K

NTILE = VOCAB // 128
TAIL_BASE = NTILE * 128
TAIL_N = VOCAB - TAIL_BASE
TS_ROWS_PAD = 1024


def _sc_body(uid_hbm, ts_hbm, tabT_hbm, tail_hbm, tts_hbm, bkt_hbm,
             consts_hbm, out_hbm, uid_v, ts_v, tidx_v, bkt_v, consts_v,
             tail_v, tts_v, blk0, blk1, blk2, blk3, blk4, blk5, blk6, blk7,
             stage_v, sem):
    wid = lax.axis_index("s") * NC + lax.axis_index("c")
    base = wid * BPW
    slot_a = [blk0, blk1, blk2, blk3]
    slot_b = [blk4, blk5, blk6, blk7]
    lanes = lax.iota(jnp.int32, L)

    pltpu.sync_copy(uid_hbm.at[pl.ds(base, BPW)], uid_v)
    pltpu.sync_copy(ts_hbm.at[pl.ds(base, BPW)], ts_v)
    pltpu.sync_copy(bkt_hbm, bkt_v)
    pltpu.sync_copy(consts_hbm, consts_v)
    pltpu.sync_copy(tail_hbm, tail_v)
    pltpu.sync_copy(tts_hbm, tts_v)

    inv_step = consts_v[pl.ds(0, L)]
    mean = consts_v[pl.ds(L, L)]
    denom = consts_v[pl.ds(2 * L, L)]

    def uscalar(j):
        voff = pl.multiple_of((j >> 4) * L, L)
        vec = uid_v[pl.ds(voff, L)]
        return jnp.sum(jnp.where(lanes == (j & (L - 1)), vec, 0))

    def fire_group(slots, gi, hb):
        us = []
        for k in range(K):
            j = jnp.minimum(gi * K + k, HALF - 1) + hb
            u = uscalar(j)
            utile = jnp.minimum(u >> 7, NTILE - 1)
            off = pl.multiple_of(utile * 128, 128)
            pltpu.async_copy(tabT_hbm.at[:, pl.ds(off, 128)], slots[k], sem)
            us.append(u)
        return tuple(us)

    def drain(slots):
        for k in range(K):
            pltpu.make_async_copy(tabT_hbm.at[:, pl.ds(0, 128)],
                                  slots[k], sem).wait()

    def extract_group(slots, us, gi):
        for k in range(K):
            j = jnp.minimum(gi * K + k, HALF - 1)
            u = us[k]
            ucol = jnp.full((L,), u & 127, jnp.int32)
            istail = u >= TAIL_BASE
            uloc = jnp.clip(u - TAIL_BASE, 0, TAIL_N - 1)
            for c in range(DIM // L):
                gn = plsc.load_gather(slots[k], [c * L + lanes, ucol])
                toff = pl.multiple_of(uloc * DIM + c * L, L)
                gt = tail_v[pl.ds(toff, L)]
                stage_v[j, pl.ds(c * L, L)] = jnp.where(istail, gt, gn)

    for h in range(2):
        hb = h * HALF

        for i in range(HVEC):
            t = ts_v[pl.ds(hb + i * L, L)]
            g = jnp.clip((t * inv_step).astype(jnp.int32), 0, NBUCKETS - 2)
            blo = plsc.load_gather(bkt_v, [g])
            bhi = plsc.load_gather(bkt_v, [g + 1])
            idx = jnp.where(t < blo, g, jnp.where(t >= bhi, g + 2, g + 1))
            tidx_v[pl.ds(i * L, L)] = idx
            plsc.store_scatter(
                stage_v, [i * L + lanes, jnp.full((L,), 2 * DIM, jnp.int32)],
                (t - mean) / denom)

        carry = fire_group(slot_a, 0, hb)

        def two_groups(t2, carry):
            ua = carry
            ub = fire_group(slot_b, 2 * t2 + 1, hb)
            drain(slot_a)
            extract_group(slot_a, ua, 2 * t2)
            un = fire_group(slot_a, 2 * t2 + 2, hb)
            drain(slot_b)
            extract_group(slot_b, ub, 2 * t2 + 1)
            return un

        carry = lax.fori_loop(0, NGRP // 2, two_groups, carry)
        drain(slot_a)

        for i in range(HVEC):
            tidx = tidx_v[pl.ds(i * L, L)]
            rows = i * L + lanes
            for d in range(DIM):
                vals = plsc.load_gather(tts_v, [tidx * DIM + d])
                plsc.store_scatter(
                    stage_v, [rows, jnp.full((L,), DIM + d, jnp.int32)], vals)

        pltpu.sync_copy(stage_v, out_hbm.at[pl.ds(base + hb, HALF)])


@jax.jit
def _run(user_id, timestamp, tabT, tail, tts, buckets_pad, consts):
    mesh = plsc.VectorSubcoreMesh(core_axis_name="c", subcore_axis_name="s")
    f = functools.partial(
        pl.kernel,
        mesh=mesh,
        compiler_params=pltpu.CompilerParams(needs_layout_passes=False),
        out_type=jax.ShapeDtypeStruct((B, OUT_PAD), jnp.float32),
        scratch_types=[
            pltpu.VMEM((BPW,), jnp.int32),
            pltpu.VMEM((BPW,), jnp.float32),
            pltpu.VMEM((HALF,), jnp.int32),
            pltpu.VMEM((BKT_PAD,), jnp.float32),
            pltpu.VMEM((CONST_PAD,), jnp.float32),
            pltpu.VMEM((TAIL_N * DIM,), jnp.float32),
            pltpu.VMEM((TS_ROWS_PAD * DIM,), jnp.float32),
        ] + [pltpu.VMEM((DIM, 128), jnp.float32)] * 8 + [
            pltpu.VMEM((HALF, OUT_PAD), jnp.float32),
            pltpu.SemaphoreType.DMA,
        ],
    )(_sc_body)
    return f(user_id, timestamp, tabT, tail, tts, buckets_pad, consts)


def kernel(user_id, timestamp, user_table, ts_table, buckets, norm_mean,
           norm_var):
    n = buckets.shape[0]
    inv_step = (n - 1.0) / (buckets[-1] - buckets[0])
    denom = jnp.sqrt(norm_var + 1e-6)
    consts = jnp.concatenate([
        jnp.full((L,), inv_step, jnp.float32),
        jnp.full((L,), norm_mean, jnp.float32),
        jnp.full((L,), denom, jnp.float32),
        jnp.zeros((CONST_PAD - 3 * L,), jnp.float32),
    ])
    buckets_pad = jnp.concatenate(
        [buckets, jnp.full((BKT_PAD - n,), jnp.inf, jnp.float32)])
    tabT = user_table.T
    tail = user_table[TAIL_BASE:].reshape(-1)
    tts = jnp.pad(
        ts_table, ((0, TS_ROWS_PAD - ts_table.shape[0]), (0, 0))).reshape(-1)
    outp = _run(user_id, timestamp, tabT, tail, tts, buckets_pad, consts)
    return outp[:, :OUT_COLS]


# speedup vs baseline: 15.9093x; 1.0738x over previous
import functools

import jax
import jax.numpy as jnp
from jax import lax
from jax.experimental import pallas as pl
from jax.experimental.pallas import tpu as pltpu
from jax.experimental.pallas import tpu_sc as plsc

B = 16384
VOCAB = 1000000
DIM = 32
NBUCKETS = 1000
OUT_COLS = 2 * DIM + 1
OUT_PAD = 72

NC = 2
NS = 16
L = 16
NW = NC * NS
BPW = B // NW
NVEC = BPW // L
BKT_PAD = 1024
CONST_PAD = 128
K = 4
NGRP = BPW // K

NTILE = VOCAB // 128
TAIL_BASE = NTILE * 128
TAIL_N = VOCAB - TAIL_BASE
TS_ROWS_PAD = 1024


def _sc_body(uid_hbm, ts_hbm, tabT_hbm, tail_hbm, tts_hbm, bkt_hbm,
             consts_hbm, out_hbm, uid_v, ts_v, tidx_v, bkt_v, consts_v,
             tail_v, tts_v, blk0, blk1, blk2, blk3, blk4, blk5, blk6, blk7,
             stage_v, sem):
    wid = lax.axis_index("s") * NC + lax.axis_index("c")
    base = wid * BPW
    slot_a = [blk0, blk1, blk2, blk3]
    slot_b = [blk4, blk5, blk6, blk7]
    lanes = lax.iota(jnp.int32, L)

    pltpu.sync_copy(uid_hbm.at[pl.ds(base, BPW)], uid_v)
    pltpu.sync_copy(ts_hbm.at[pl.ds(base, BPW)], ts_v)
    pltpu.sync_copy(bkt_hbm, bkt_v)
    pltpu.sync_copy(consts_hbm, consts_v)
    pltpu.sync_copy(tail_hbm, tail_v)
    pltpu.sync_copy(tts_hbm, tts_v)

    inv_step = consts_v[pl.ds(0, L)]
    mean = consts_v[pl.ds(L, L)]
    denom = consts_v[pl.ds(2 * L, L)]

    def uscalar(j):
        voff = pl.multiple_of((j >> 4) * L, L)
        vec = uid_v[pl.ds(voff, L)]
        return jnp.sum(jnp.where(lanes == (j & (L - 1)), vec, 0))

    def fire_group(slots, gi, hb):
        us = []
        for k in range(K):
            j = jnp.minimum(gi * K + k, BPW - 1) + hb
            u = uscalar(j)
            utile = jnp.minimum(u >> 7, NTILE - 1)
            off = pl.multiple_of(utile * 128, 128)
            pltpu.async_copy(tabT_hbm.at[:, pl.ds(off, 128)], slots[k], sem)
            us.append(u)
        return tuple(us)

    def drain(slots):
        for k in range(K):
            pltpu.make_async_copy(tabT_hbm.at[:, pl.ds(0, 128)],
                                  slots[k], sem).wait()

    def extract_group(slots, us, gi):
        for k in range(K):
            j = jnp.minimum(gi * K + k, BPW - 1)
            u = us[k]
            ucol = jnp.full((L,), u & 127, jnp.int32)
            istail = u >= TAIL_BASE
            uloc = jnp.clip(u - TAIL_BASE, 0, TAIL_N - 1)
            jcol = jnp.full((L,), j, jnp.int32)
            for c in range(DIM // L):
                gn = plsc.load_gather(slots[k], [c * L + lanes, ucol])
                toff = pl.multiple_of(uloc * DIM + c * L, L)
                gt = tail_v[pl.ds(toff, L)]
                plsc.store_scatter(stage_v, [c * L + lanes, jcol],
                                   jnp.where(istail, gt, gn))

    for i in range(NVEC):
        t = ts_v[pl.ds(i * L, L)]
        g = jnp.clip((t * inv_step).astype(jnp.int32), 0, NBUCKETS - 2)
        blo = plsc.load_gather(bkt_v, [g])
        bhi = plsc.load_gather(bkt_v, [g + 1])
        idx = jnp.where(t < blo, g, jnp.where(t >= bhi, g + 2, g + 1))
        tidx_v[pl.ds(i * L, L)] = idx
        stage_v[2 * DIM, pl.ds(i * L, L)] = (t - mean) / denom

    carry = fire_group(slot_a, 0, 0)

    def two_groups(t2, carry):
        ua = carry
        ub = fire_group(slot_b, 2 * t2 + 1, 0)
        drain(slot_a)
        extract_group(slot_a, ua, 2 * t2)
        un = fire_group(slot_a, 2 * t2 + 2, 0)
        drain(slot_b)
        extract_group(slot_b, ub, 2 * t2 + 1)
        return un

    carry = lax.fori_loop(0, NGRP // 2, two_groups, carry)
    drain(slot_a)

    for i in range(NVEC):
        tidx = tidx_v[pl.ds(i * L, L)]
        for d in range(DIM):
            vals = plsc.load_gather(tts_v, [tidx * DIM + d])
            stage_v[DIM + d, pl.ds(i * L, L)] = vals

    pltpu.sync_copy(stage_v, out_hbm.at[:, pl.ds(base, BPW)])


@jax.jit
def _run(user_id, timestamp, tabT, tail, tts, buckets_pad, consts):
    mesh = plsc.VectorSubcoreMesh(core_axis_name="c", subcore_axis_name="s")
    f = functools.partial(
        pl.kernel,
        mesh=mesh,
        compiler_params=pltpu.CompilerParams(needs_layout_passes=False),
        out_type=jax.ShapeDtypeStruct((OUT_PAD, B), jnp.float32),
        scratch_types=[
            pltpu.VMEM((BPW,), jnp.int32),
            pltpu.VMEM((BPW,), jnp.float32),
            pltpu.VMEM((BPW,), jnp.int32),
            pltpu.VMEM((BKT_PAD,), jnp.float32),
            pltpu.VMEM((CONST_PAD,), jnp.float32),
            pltpu.VMEM((TAIL_N * DIM,), jnp.float32),
            pltpu.VMEM((TS_ROWS_PAD * DIM,), jnp.float32),
        ] + [pltpu.VMEM((DIM, 128), jnp.float32)] * 8 + [
            pltpu.VMEM((OUT_PAD, BPW), jnp.float32),
            pltpu.SemaphoreType.DMA,
        ],
    )(_sc_body)
    return f(user_id, timestamp, tabT, tail, tts, buckets_pad, consts)


def kernel(user_id, timestamp, user_table, ts_table, buckets, norm_mean,
           norm_var):
    n = buckets.shape[0]
    inv_step = (n - 1.0) / (buckets[-1] - buckets[0])
    denom = jnp.sqrt(norm_var + 1e-6)
    consts = jnp.concatenate([
        jnp.full((L,), inv_step, jnp.float32),
        jnp.full((L,), norm_mean, jnp.float32),
        jnp.full((L,), denom, jnp.float32),
        jnp.zeros((CONST_PAD - 3 * L,), jnp.float32),
    ])
    buckets_pad = jnp.concatenate(
        [buckets, jnp.full((BKT_PAD - n,), jnp.inf, jnp.float32)])
    tabT = user_table.T
    tail = user_table[TAIL_BASE:].reshape(-1)
    tts = jnp.pad(
        ts_table, ((0, TS_ROWS_PAD - ts_table.shape[0]), (0, 0))).reshape(-1)
    outp = _run(user_id, timestamp, tabT, tail, tts, buckets_pad, consts)
    return outp[:OUT_COLS].T
